# baseline (speedup 1.0000x reference)
import jax
import jax.numpy as jnp
from jax import lax
from jax.experimental import pallas as pl
from jax.experimental.pallas import tpu as pltpu, tpu_sc as plsc

NC = 2
NS = 16
L = 16
NW = NC * NS
B = 16384
F = 16
BPW = B // NW
CHUNK = 128
NCHUNK = BPW // CHUNK
NBLK = BPW // L


def _sc_body(u_hbm, i_hbm, p_hbm, q_hbm, bu_hbm, bi_hbm, out_hbm,
             uidx_v, iidx_v, prow_v, qrow_v, bu_v, bi_v, out_v, sem):
    wid = lax.axis_index("s") * NC + lax.axis_index("c")
    base = wid * BPW
    pltpu.sync_copy(u_hbm.at[pl.ds(base, BPW)], uidx_v)
    pltpu.sync_copy(i_hbm.at[pl.ds(base, BPW)], iidx_v)

    descs = []
    for j in range(NCHUNK):
        s = pl.ds(j * CHUNK, CHUNK)
        descs.append(pltpu.async_copy(p_hbm.at[uidx_v.at[s]], prow_v.at[s], sem))
        descs.append(pltpu.async_copy(q_hbm.at[iidx_v.at[s]], qrow_v.at[s], sem))
        descs.append(pltpu.async_copy(bu_hbm.at[uidx_v.at[s]], bu_v.at[s], sem))
        descs.append(pltpu.async_copy(bi_hbm.at[iidx_v.at[s]], bi_v.at[s], sem))
    for d in descs:
        d.wait()

    lane = lax.iota(jnp.int32, L)

    def blk(k, carry):
        base = k * L
        acc = bu_v[pl.ds(base, L)] + bi_v[pl.ds(base, L)]
        for j in range(L):
            p = prow_v[base + j]
            q = qrow_v[base + j]
            s = jnp.sum(p * q)
            acc = jnp.where(lane == j, acc + s, acc)
        out_v[pl.ds(base, L)] = acc
        return carry

    lax.fori_loop(0, NBLK, blk, 0)
    pltpu.sync_copy(out_v, out_hbm.at[pl.ds(base, BPW)])


def kernel(user_id, item_id, P, Q, Bu, Bi):
    u = user_id.reshape(-1)
    i = item_id.reshape(-1)
    bu = Bu.reshape(-1)
    bi = Bi.reshape(-1)
    mesh = plsc.VectorSubcoreMesh(core_axis_name="c", subcore_axis_name="s",
                                  num_cores=NC, num_subcores=NS)
    out = pl.kernel(
        _sc_body,
        out_type=jax.ShapeDtypeStruct((B,), jnp.float32),
        mesh=mesh,
        compiler_params=pltpu.CompilerParams(needs_layout_passes=False,
                                             use_tc_tiling_on_sc=False),
        scratch_types=[
            pltpu.VMEM((BPW,), jnp.int32),
            pltpu.VMEM((BPW,), jnp.int32),
            pltpu.VMEM((BPW, F), jnp.float32),
            pltpu.VMEM((BPW, F), jnp.float32),
            pltpu.VMEM((BPW,), jnp.float32),
            pltpu.VMEM((BPW,), jnp.float32),
            pltpu.VMEM((BPW,), jnp.float32),
            pltpu.SemaphoreType.DMA,
        ],
    )(u, i, P, Q, bu, bi)
    return out.reshape(B, 1)


# speedup vs baseline: 3.7393x; 3.7393x over previous
import jax
import jax.numpy as jnp
from jax import lax
from jax.experimental import pallas as pl
from jax.experimental.pallas import tpu as pltpu, tpu_sc as plsc

NC = 2
NS = 16
L = 16
NW = NC * NS
B = 16384
F = 16
BPW = B // NW
CHUNK = 128
NCHUNK = BPW // CHUNK
G = 8
NG = BPW // G


def _sc_body(u_hbm, i_hbm, pt_hbm, qt_hbm, bu_hbm, bi_hbm, out_hbm,
             uidx_v, iidx_v, pbuf_v, qbuf_v, bu_v, bi_v, out_v,
             sem0, sem1, gsem):
    wid = lax.axis_index("s") * NC + lax.axis_index("c")
    base = wid * BPW
    pltpu.sync_copy(u_hbm.at[pl.ds(base, BPW)], uidx_v.at[pl.ds(0, BPW)])
    pltpu.sync_copy(i_hbm.at[pl.ds(base, BPW)], iidx_v.at[pl.ds(0, BPW)])

    gdescs = []
    for j in range(NCHUNK):
        s = pl.ds(j * CHUNK, CHUNK)
        gdescs.append(pltpu.async_copy(bu_hbm.at[uidx_v.at[s]], bu_v.at[s], gsem))
        gdescs.append(pltpu.async_copy(bi_hbm.at[iidx_v.at[s]], bi_v.at[s], gsem))

    lane = lax.iota(jnp.int32, L)
    sems = (sem0, sem1)

    def fire(g, par):
        uv = uidx_v[pl.ds(g * G, L)]
        iv = iidx_v[pl.ds(g * G, L)]
        sem = sems[0] if par == 0 else sems[1]
        for j in range(G):
            cu = pl.multiple_of((uv[j] >> 7) * 128, 128)
            ci = pl.multiple_of((iv[j] >> 7) * 128, 128)
            pltpu.async_copy(pt_hbm.at[:, pl.ds(cu, 128)], pbuf_v.at[par, j], sem)
            pltpu.async_copy(qt_hbm.at[:, pl.ds(ci, 128)], qbuf_v.at[par, j], sem)

    def proc(g, par, half, acc):
        uv = uidx_v[pl.ds(g * G, L)]
        iv = iidx_v[pl.ds(g * G, L)]
        sem = sems[par]
        for j in range(G):
            pltpu.make_async_copy(pt_hbm.at[:, pl.ds(0, 128)],
                                  pbuf_v.at[par, j], sem).wait()
            pltpu.make_async_copy(qt_hbm.at[:, pl.ds(0, 128)],
                                  qbuf_v.at[par, j], sem).wait()
            lu = jnp.full((L,), uv[j] & 127, jnp.int32)
            li = jnp.full((L,), iv[j] & 127, jnp.int32)
            pv = plsc.load_gather(pbuf_v.at[par, j], [lane, lu])
            qv = plsc.load_gather(qbuf_v.at[par, j], [lane, li])
            s = jnp.sum(pv * qv)
            acc = jnp.where(lane == half + j, acc + s, acc)
        return acc

    for d in gdescs:
        d.wait()

    fire(0, 0)

    def pair(k, carry):
        g0 = 2 * k
        g1 = g0 + 1
        fire(g1, 1)
        blk = pl.ds(k * L, L)
        acc = bu_v[blk] + bi_v[blk]
        acc = proc(g0, 0, 0, acc)

        @pl.when(g0 + 2 < NG)
        def _():
            fire(g0 + 2, 0)

        acc = proc(g1, 1, G, acc)
        out_v[blk] = acc
        return carry

    lax.fori_loop(0, NG // 2, pair, 0)
    pltpu.sync_copy(out_v, out_hbm.at[pl.ds(base, BPW)])


def kernel(user_id, item_id, P, Q, Bu, Bi):
    u = user_id.reshape(-1)
    i = item_id.reshape(-1)
    bu = Bu.reshape(-1)
    bi = Bi.reshape(-1)
    pt = P.T
    qt = Q.T
    mesh = plsc.VectorSubcoreMesh(core_axis_name="c", subcore_axis_name="s",
                                  num_cores=NC, num_subcores=NS)
    out = pl.kernel(
        _sc_body,
        out_type=jax.ShapeDtypeStruct((B,), jnp.float32),
        mesh=mesh,
        compiler_params=pltpu.CompilerParams(needs_layout_passes=False),
        scratch_types=[
            pltpu.VMEM((BPW + L,), jnp.int32),
            pltpu.VMEM((BPW + L,), jnp.int32),
            pltpu.VMEM((2, G, F, 128), jnp.float32),
            pltpu.VMEM((2, G, F, 128), jnp.float32),
            pltpu.VMEM((BPW,), jnp.float32),
            pltpu.VMEM((BPW,), jnp.float32),
            pltpu.VMEM((BPW,), jnp.float32),
            pltpu.SemaphoreType.DMA,
            pltpu.SemaphoreType.DMA,
            pltpu.SemaphoreType.DMA,
        ],
    )(u, i, pt, qt, bu, bi)
    return out.reshape(B, 1)
